# baseline (speedup 1.0000x reference)
import functools

import jax
import jax.numpy as jnp
from jax import lax
from jax.experimental import pallas as pl
from jax.experimental.pallas import tpu as pltpu
from jax.experimental.pallas import tpu_sc as plsc


def _sc_gather(table, ticker_ids):
    info = plsc.get_sparse_core_info()
    nw = info.num_cores * info.num_subcores
    b = ticker_ids.shape[0]
    d = table.shape[1]
    b_per_w = b // nw
    mesh = plsc.VectorSubcoreMesh(core_axis_name="c", subcore_axis_name="s")

    @functools.partial(
        pl.kernel,
        mesh=mesh,
        out_type=jax.ShapeDtypeStruct((b, d), jnp.float32),
        scratch_types=[
            pltpu.VMEM((16,), jnp.int32),
            pltpu.VMEM((b_per_w, 8, d), jnp.float32),
            pltpu.VMEM((b_per_w, d), jnp.float32),
            pltpu.SemaphoreType.DMA,
        ],
        compiler_params=pltpu.CompilerParams(needs_layout_passes=False),
    )
    def gather_kernel(table_hbm, idx_hbm, out_hbm, idx_v, tiles_v, rows_v, sem):
        wid = lax.axis_index("s") * info.num_cores + lax.axis_index("c")
        base = wid * b_per_w
        pltpu.sync_copy(idx_hbm.at[pl.ds(base, b_per_w)], idx_v.at[pl.ds(0, b_per_w)])
        iv = idx_v[...]
        copies = []
        for i in range(b_per_w):
            grp = (iv[i] // 8) * 8
            copies.append(
                pltpu.async_copy(table_hbm.at[pl.ds(grp, 8), :], tiles_v.at[i], sem)
            )
        for c in copies:
            c.wait()
        lane = lax.iota(jnp.int32, 16)
        for i in range(b_per_w):
            i_vec = jnp.broadcast_to(i, (16,)).astype(jnp.int32)
            r_vec = jnp.broadcast_to(iv[i] % 8, (16,))
            for j in range(d // 16):
                vals = plsc.load_gather(tiles_v, [i_vec, r_vec, lane + 16 * j])
                rows_v[i, pl.ds(16 * j, 16)] = vals
        pltpu.sync_copy(rows_v, out_hbm.at[pl.ds(base, b_per_w)])

    return gather_kernel(table, ticker_ids)


def _tc_broadcast_add(x, emb):
    B, T, S, D = x.shape
    G = 32

    def body(x_ref, e_ref, o_ref):
        o_ref[...] = x_ref[...] + e_ref[...][None, :, None, :]

    return pl.pallas_call(
        body,
        grid=(B, T // G),
        in_specs=[
            pl.BlockSpec((1, G, S, D), lambda b, t: (b, t, 0, 0)),
            pl.BlockSpec((G, D), lambda b, t: (t, 0)),
        ],
        out_specs=pl.BlockSpec((1, G, S, D), lambda b, t: (b, t, 0, 0)),
        out_shape=jax.ShapeDtypeStruct(x.shape, x.dtype),
    )(x, emb)


def kernel(x, ticker_ids, table):
    emb = jnp.zeros((ticker_ids.shape[0], table.shape[1]), jnp.float32)
    return _tc_broadcast_add(x, emb)


# speedup vs baseline: 3.2035x; 3.2035x over previous
import functools

import jax
import jax.numpy as jnp
from jax import lax
from jax.experimental import pallas as pl
from jax.experimental.pallas import tpu as pltpu
from jax.experimental.pallas import tpu_sc as plsc


def _sc_gather(table_t, ticker_ids):
    info = plsc.get_sparse_core_info()
    nw = info.num_cores * info.num_subcores
    b = ticker_ids.shape[0]
    d = table_t.shape[0]
    b_per_w = b // nw
    mesh = plsc.VectorSubcoreMesh(core_axis_name="c", subcore_axis_name="s")

    @functools.partial(
        pl.kernel,
        mesh=mesh,
        out_type=jax.ShapeDtypeStruct((b, d), jnp.float32),
        scratch_types=[
            pltpu.VMEM((16,), jnp.int32),
            pltpu.VMEM((b_per_w, d, 128), jnp.float32),
            pltpu.VMEM((b_per_w, d), jnp.float32),
            pltpu.SemaphoreType.DMA,
        ],
        compiler_params=pltpu.CompilerParams(needs_layout_passes=False),
    )
    def gather_kernel(table_hbm, idx_hbm, out_hbm, idx_v, tiles_v, rows_v, sem):
        wid = lax.axis_index("s") * info.num_cores + lax.axis_index("c")
        base = wid * b_per_w
        pltpu.sync_copy(idx_hbm.at[pl.ds(base, b_per_w)], idx_v.at[pl.ds(0, b_per_w)])
        iv = idx_v[...]
        copies = []
        for i in range(b_per_w):
            grp = (iv[i] // 128) * 128
            copies.append(
                pltpu.async_copy(table_hbm.at[:, pl.ds(grp, 128)], tiles_v.at[i], sem)
            )
        for c in copies:
            c.wait()
        lane = lax.iota(jnp.int32, 16)
        for i in range(b_per_w):
            i_vec = jnp.broadcast_to(i, (16,)).astype(jnp.int32)
            c_vec = jnp.broadcast_to(iv[i] % 128, (16,))
            for j in range(d // 16):
                vals = plsc.load_gather(tiles_v, [i_vec, lane + 16 * j, c_vec])
                rows_v[i, pl.ds(16 * j, 16)] = vals
        pltpu.sync_copy(rows_v, out_hbm.at[pl.ds(base, b_per_w)])

    return gather_kernel(table_t, ticker_ids)


def _tc_broadcast_add(x_t, emb_t):
    B, S, D, T = x_t.shape
    SG = 25

    def body(x_ref, e_ref, o_ref):
        o_ref[...] = x_ref[...] + e_ref[...][None, None, :, :]

    return pl.pallas_call(
        body,
        grid=(B, S // SG),
        in_specs=[
            pl.BlockSpec((1, SG, D, T), lambda b, s: (b, s, 0, 0)),
            pl.BlockSpec((D, T), lambda b, s: (0, 0)),
        ],
        out_specs=pl.BlockSpec((1, SG, D, T), lambda b, s: (b, s, 0, 0)),
        out_shape=jax.ShapeDtypeStruct(x_t.shape, x_t.dtype),
    )(x_t, emb_t)


def kernel(x, ticker_ids, table):
    x_t = jnp.transpose(x, (0, 2, 3, 1))
    table_t = jnp.transpose(table, (1, 0))
    emb = _sc_gather(table_t, ticker_ids)
    emb_t = jnp.transpose(emb, (1, 0))
    out_t = _tc_broadcast_add(x_t, emb_t)
    return jnp.transpose(out_t, (0, 3, 1, 2))


# speedup vs baseline: 3.5118x; 1.0962x over previous
import functools

import jax
import jax.numpy as jnp
from jax import lax
from jax.experimental import pallas as pl
from jax.experimental.pallas import tpu as pltpu
from jax.experimental.pallas import tpu_sc as plsc


def _sc_gather(table_t, ticker_ids):
    info = plsc.get_sparse_core_info()
    nw = info.num_cores * info.num_subcores
    b = ticker_ids.shape[0]
    d = table_t.shape[0]
    b_per_w = b // nw
    mesh = plsc.VectorSubcoreMesh(core_axis_name="c", subcore_axis_name="s")

    @functools.partial(
        pl.kernel,
        mesh=mesh,
        out_type=jax.ShapeDtypeStruct((b, d), jnp.float32),
        scratch_types=[
            pltpu.VMEM((16,), jnp.int32),
            pltpu.VMEM((b_per_w, d, 128), jnp.float32),
            pltpu.VMEM((b_per_w, d), jnp.float32),
            pltpu.SemaphoreType.DMA,
        ],
        compiler_params=pltpu.CompilerParams(needs_layout_passes=False),
    )
    def gather_kernel(table_hbm, idx_hbm, out_hbm, idx_v, tiles_v, rows_v, sem):
        wid = lax.axis_index("s") * info.num_cores + lax.axis_index("c")
        base = wid * b_per_w
        pltpu.sync_copy(idx_hbm.at[pl.ds(base, b_per_w)], idx_v.at[pl.ds(0, b_per_w)])
        iv = idx_v[...]
        copies = []
        for i in range(b_per_w):
            grp = (iv[i] // 128) * 128
            copies.append(
                pltpu.async_copy(table_hbm.at[:, pl.ds(grp, 128)], tiles_v.at[i], sem)
            )
        for c in copies:
            c.wait()
        lane = lax.iota(jnp.int32, 16)
        for i in range(b_per_w):
            i_vec = jnp.broadcast_to(i, (16,)).astype(jnp.int32)
            c_vec = jnp.broadcast_to(iv[i] % 128, (16,))
            for j in range(d // 16):
                vals = plsc.load_gather(tiles_v, [i_vec, lane + 16 * j, c_vec])
                rows_v[i, pl.ds(16 * j, 16)] = vals
        pltpu.sync_copy(rows_v, out_hbm.at[pl.ds(base, b_per_w)])

    return gather_kernel(table_t, ticker_ids)


def _tc_broadcast_add(x_t, emb_t):
    B, S, D, T = x_t.shape
    SG = 50

    def body(x_ref, e_ref, o_ref):
        o_ref[...] = x_ref[...] + e_ref[...][None, None, :, :]

    return pl.pallas_call(
        body,
        grid=(B, S // SG),
        in_specs=[
            pl.BlockSpec((1, SG, D, T), lambda b, s: (b, s, 0, 0)),
            pl.BlockSpec((D, T), lambda b, s: (0, 0)),
        ],
        out_specs=pl.BlockSpec((1, SG, D, T), lambda b, s: (b, s, 0, 0)),
        out_shape=jax.ShapeDtypeStruct(x_t.shape, x_t.dtype),
    )(x_t, emb_t)


def kernel(x, ticker_ids, table):
    x_t = jnp.transpose(x, (0, 2, 3, 1))
    table_t = jnp.transpose(table, (1, 0))
    emb = _sc_gather(table_t, ticker_ids)
    emb_t = jnp.transpose(emb, (1, 0))
    out_t = _tc_broadcast_add(x_t, emb_t)
    return jnp.transpose(out_t, (0, 3, 1, 2))


# speedup vs baseline: 3.6050x; 1.0265x over previous
import functools

import jax
import jax.numpy as jnp
from jax import lax
from jax.experimental import pallas as pl
from jax.experimental.pallas import tpu as pltpu
from jax.experimental.pallas import tpu_sc as plsc


def _sc_gather(table_t, ticker_ids):
    info = plsc.get_sparse_core_info()
    nw = info.num_cores * info.num_subcores
    b = ticker_ids.shape[0]
    d = table_t.shape[0]
    b_per_w = b // nw
    mesh = plsc.VectorSubcoreMesh(core_axis_name="c", subcore_axis_name="s")

    @functools.partial(
        pl.kernel,
        mesh=mesh,
        out_type=jax.ShapeDtypeStruct((b, d), jnp.float32),
        scratch_types=[
            pltpu.VMEM((16,), jnp.int32),
            pltpu.VMEM((b_per_w, d, 128), jnp.float32),
            pltpu.VMEM((b_per_w, d), jnp.float32),
            pltpu.SemaphoreType.DMA,
        ],
        compiler_params=pltpu.CompilerParams(needs_layout_passes=False),
    )
    def gather_kernel(table_hbm, idx_hbm, out_hbm, idx_v, tiles_v, rows_v, sem):
        wid = lax.axis_index("s") * info.num_cores + lax.axis_index("c")
        base = wid * b_per_w
        pltpu.sync_copy(idx_hbm.at[pl.ds(base, b_per_w)], idx_v.at[pl.ds(0, b_per_w)])
        iv = idx_v[...]
        copies = []
        for i in range(b_per_w):
            grp = (iv[i] // 128) * 128
            copies.append(
                pltpu.async_copy(table_hbm.at[:, pl.ds(grp, 128)], tiles_v.at[i], sem)
            )
        for c in copies:
            c.wait()
        lane = lax.iota(jnp.int32, 16)
        for i in range(b_per_w):
            i_vec = jnp.broadcast_to(i, (16,)).astype(jnp.int32)
            c_vec = jnp.broadcast_to(iv[i] % 128, (16,))
            for j in range(d // 16):
                vals = plsc.load_gather(tiles_v, [i_vec, lane + 16 * j, c_vec])
                rows_v[i, pl.ds(16 * j, 16)] = vals
        pltpu.sync_copy(rows_v, out_hbm.at[pl.ds(base, b_per_w)])

    return gather_kernel(table_t, ticker_ids)


def _tc_broadcast_add(x_t, emb_t):
    B, S, D, T = x_t.shape
    SG = 100

    def body(x_ref, e_ref, o_ref):
        o_ref[...] = x_ref[...] + e_ref[...][None, None, :, :]

    return pl.pallas_call(
        body,
        grid=(B, S // SG),
        in_specs=[
            pl.BlockSpec((1, SG, D, T), lambda b, s: (b, s, 0, 0)),
            pl.BlockSpec((D, T), lambda b, s: (0, 0)),
        ],
        out_specs=pl.BlockSpec((1, SG, D, T), lambda b, s: (b, s, 0, 0)),
        out_shape=jax.ShapeDtypeStruct(x_t.shape, x_t.dtype),
    )(x_t, emb_t)


def kernel(x, ticker_ids, table):
    x_t = jnp.transpose(x, (0, 2, 3, 1))
    table_t = jnp.transpose(table, (1, 0))
    emb = _sc_gather(table_t, ticker_ids)
    emb_t = jnp.transpose(emb, (1, 0))
    out_t = _tc_broadcast_add(x_t, emb_t)
    return jnp.transpose(out_t, (0, 3, 1, 2))


# speedup vs baseline: 3.6667x; 1.0171x over previous
import functools

import jax
import jax.numpy as jnp
from jax import lax
from jax.experimental import pallas as pl
from jax.experimental.pallas import tpu as pltpu
from jax.experimental.pallas import tpu_sc as plsc


def _sc_gather(table_t, ticker_ids):
    info = plsc.get_sparse_core_info()
    nw = info.num_cores * info.num_subcores
    b = ticker_ids.shape[0]
    d = table_t.shape[0]
    b_per_w = b // nw
    mesh = plsc.VectorSubcoreMesh(core_axis_name="c", subcore_axis_name="s")

    @functools.partial(
        pl.kernel,
        mesh=mesh,
        out_type=jax.ShapeDtypeStruct((b, d), jnp.float32),
        scratch_types=[
            pltpu.VMEM((16,), jnp.int32),
            pltpu.VMEM((b_per_w, d, 128), jnp.float32),
            pltpu.VMEM((b_per_w, d), jnp.float32),
            pltpu.SemaphoreType.DMA,
        ],
        compiler_params=pltpu.CompilerParams(needs_layout_passes=False),
    )
    def gather_kernel(table_hbm, idx_hbm, out_hbm, idx_v, tiles_v, rows_v, sem):
        wid = lax.axis_index("s") * info.num_cores + lax.axis_index("c")
        base = wid * b_per_w
        pltpu.sync_copy(idx_hbm.at[pl.ds(base, b_per_w)], idx_v.at[pl.ds(0, b_per_w)])
        iv = idx_v[...]
        copies = []
        for i in range(b_per_w):
            grp = (iv[i] // 128) * 128
            copies.append(
                pltpu.async_copy(table_hbm.at[:, pl.ds(grp, 128)], tiles_v.at[i], sem)
            )
        for c in copies:
            c.wait()
        lane = lax.iota(jnp.int32, 16)
        for i in range(b_per_w):
            i_vec = jnp.broadcast_to(i, (16,)).astype(jnp.int32)
            c_vec = jnp.broadcast_to(iv[i] % 128, (16,))
            for j in range(d // 16):
                vals = plsc.load_gather(tiles_v, [i_vec, lane + 16 * j, c_vec])
                rows_v[i, pl.ds(16 * j, 16)] = vals
        pltpu.sync_copy(rows_v, out_hbm.at[pl.ds(base, b_per_w)])

    return gather_kernel(table_t, ticker_ids)


def _tc_broadcast_add(x_t, emb_t):
    B, S, D, T = x_t.shape
    SG = 200

    def body(x_ref, e_ref, o_ref):
        o_ref[...] = x_ref[...] + e_ref[...][None, None, :, :]

    return pl.pallas_call(
        body,
        grid=(B, S // SG),
        in_specs=[
            pl.BlockSpec((1, SG, D, T), lambda b, s: (b, s, 0, 0)),
            pl.BlockSpec((D, T), lambda b, s: (0, 0)),
        ],
        out_specs=pl.BlockSpec((1, SG, D, T), lambda b, s: (b, s, 0, 0)),
        out_shape=jax.ShapeDtypeStruct(x_t.shape, x_t.dtype),
    )(x_t, emb_t)


def kernel(x, ticker_ids, table):
    x_t = jnp.transpose(x, (0, 2, 3, 1))
    table_t = jnp.transpose(table, (1, 0))
    emb = _sc_gather(table_t, ticker_ids)
    emb_t = jnp.transpose(emb, (1, 0))
    out_t = _tc_broadcast_add(x_t, emb_t)
    return jnp.transpose(out_t, (0, 3, 1, 2))


# speedup vs baseline: 3.7619x; 1.0260x over previous
import functools

import jax
import jax.numpy as jnp
from jax import lax
from jax.experimental import pallas as pl
from jax.experimental.pallas import tpu as pltpu
from jax.experimental.pallas import tpu_sc as plsc


def _sc_gather(table_t, ticker_ids):
    info = plsc.get_sparse_core_info()
    nw = info.num_cores * info.num_subcores
    b = ticker_ids.shape[0]
    d = table_t.shape[0]
    b_per_w = b // nw
    mesh = plsc.VectorSubcoreMesh(core_axis_name="c", subcore_axis_name="s")

    @functools.partial(
        pl.kernel,
        mesh=mesh,
        out_type=jax.ShapeDtypeStruct((b, d), jnp.float32),
        scratch_types=[
            pltpu.VMEM((16,), jnp.int32),
            pltpu.VMEM((b_per_w, d, 128), jnp.float32),
            pltpu.VMEM((b_per_w, d), jnp.float32),
            pltpu.SemaphoreType.DMA,
        ],
        compiler_params=pltpu.CompilerParams(needs_layout_passes=False),
    )
    def gather_kernel(table_hbm, idx_hbm, out_hbm, idx_v, tiles_v, rows_v, sem):
        wid = lax.axis_index("s") * info.num_cores + lax.axis_index("c")
        base = wid * b_per_w
        pltpu.sync_copy(idx_hbm.at[pl.ds(base, b_per_w)], idx_v.at[pl.ds(0, b_per_w)])
        iv = idx_v[...]
        copies = []
        for i in range(b_per_w):
            grp = (iv[i] // 128) * 128
            copies.append(
                pltpu.async_copy(table_hbm.at[:, pl.ds(grp, 128)], tiles_v.at[i], sem)
            )
        for c in copies:
            c.wait()
        lane = lax.iota(jnp.int32, 16)
        for i in range(b_per_w):
            i_vec = jnp.broadcast_to(i, (16,)).astype(jnp.int32)
            c_vec = jnp.broadcast_to(iv[i] % 128, (16,))
            for j in range(d // 16):
                vals = plsc.load_gather(tiles_v, [i_vec, lane + 16 * j, c_vec])
                rows_v[i, pl.ds(16 * j, 16)] = vals
        pltpu.sync_copy(rows_v, out_hbm.at[pl.ds(base, b_per_w)])

    return gather_kernel(table_t, ticker_ids)


def _tc_broadcast_add(x_t, emb):
    B, S, D, T = x_t.shape
    SG = 200

    def body(x_ref, e_ref, o_ref):
        o_ref[...] = x_ref[...] + e_ref[...].T[None, None, :, :]

    return pl.pallas_call(
        body,
        grid=(B, S // SG),
        in_specs=[
            pl.BlockSpec((1, SG, D, T), lambda b, s: (b, s, 0, 0)),
            pl.BlockSpec((T, D), lambda b, s: (0, 0)),
        ],
        out_specs=pl.BlockSpec((1, SG, D, T), lambda b, s: (b, s, 0, 0)),
        out_shape=jax.ShapeDtypeStruct(x_t.shape, x_t.dtype),
    )(x_t, emb)


def kernel(x, ticker_ids, table):
    x_t = jnp.transpose(x, (0, 2, 3, 1))
    table_t = jnp.transpose(table, (1, 0))
    emb = _sc_gather(table_t, ticker_ids)
    out_t = _tc_broadcast_add(x_t, emb)
    return jnp.transpose(out_t, (0, 3, 1, 2))
